# baseline (speedup 1.0000x reference)
import jax
import jax.numpy as jnp
from jax import lax
from jax.experimental import pallas as pl

_NUM_EMB = 1024
_DIM = 64
_ROWS = 9216
_ROW_BLOCK = 3072
_CHUNK = 128


def _fused_body(z_ref, w_ref, idx_ref, zq_ref):
    zb = z_ref[...]
    wb = w_ref[...]
    mm2 = lax.dot_general(zb + zb, wb, (((1,), (1,)), ((), ())))
    sumz = jnp.sum(zb * zb, axis=1, keepdims=True)
    sw = jnp.sum(wb * wb, axis=1)
    chunk = _CHUNK
    m_run = (sumz - mm2[:, :chunk]) + sw[None, :chunk]
    id_run = jnp.zeros_like(m_run)
    for c in range(1, _NUM_EMB // chunk):
        d = (sumz - mm2[:, c * chunk:(c + 1) * chunk]) + sw[None, c * chunk:(c + 1) * chunk]
        upd = d < m_run
        m_run = jnp.where(upd, d, m_run)
        id_run = jnp.where(upd, float(c), id_run)
    mins = jnp.min(m_run, axis=1, keepdims=True)
    lane = lax.broadcasted_iota(jnp.int32, m_run.shape, 1).astype(jnp.float32)
    colidx = id_run * float(chunk) + lane
    cand = jnp.where(m_run == mins, colidx, float(_NUM_EMB))
    idxf = jnp.min(cand, axis=1, keepdims=True)
    idxi = idxf.astype(jnp.int32)
    onehot = jnp.where(
        lax.broadcasted_iota(jnp.int32, (_ROW_BLOCK, _NUM_EMB), 1) == idxi,
        1.0, 0.0)
    zq_ref[...] = lax.dot_general(onehot, wb, (((1,), (0,)), ((), ())))
    col = lax.broadcasted_iota(jnp.int32, (2, _NUM_EMB), 1)
    hilo = jnp.where(lax.broadcasted_iota(jnp.int32, (2, _NUM_EMB), 0) == 0,
                     col // 32, col % 32).astype(jnp.float32)
    idxrow = lax.dot_general(hilo, onehot, (((1,), (1,)), ((), ())))
    idx_ref[...] = (idxrow[0] * 32.0 + idxrow[1]).astype(jnp.int32)


def _tc_fused(zf, W, interpret=False):
    grid = _ROWS // _ROW_BLOCK
    return pl.pallas_call(
        _fused_body,
        grid=(grid,),
        in_specs=[
            pl.BlockSpec((_ROW_BLOCK, _DIM), lambda i: (i, 0)),
            pl.BlockSpec((_NUM_EMB, _DIM), lambda i: (0, 0)),
        ],
        out_specs=[
            pl.BlockSpec((_ROW_BLOCK,), lambda i: (i,)),
            pl.BlockSpec((_ROW_BLOCK, _DIM), lambda i: (i, 0)),
        ],
        out_shape=[
            jax.ShapeDtypeStruct((_ROWS,), jnp.int32),
            jax.ShapeDtypeStruct((_ROWS, _DIM), jnp.float32),
        ],
        interpret=interpret,
    )(zf, W)


def kernel(z, W):
    zf = z.reshape(-1, _DIM)
    idx, zq = _tc_fused(zf, W)
    return zq.reshape(z.shape), idx
